# baseline (speedup 1.0000x reference)
import jax
import jax.numpy as jnp
import numpy as np
from jax import lax
from jax.experimental import pallas as pl
from jax.experimental.pallas import tpu as pltpu
from jax.experimental.pallas import tpu_sc as plsc

_MAX_REL = 7
_NUM_BUCKETS = 2 * _MAX_REL + 1
_NUM_HEADS = 16
_NUM_TILES = 32
_IDX_LEN = 128


def _host_indices():
    idx_r = np.full((_NUM_TILES, _IDX_LEN), _MAX_REL, np.int32)
    idx_f = np.full((_NUM_TILES, _IDX_LEN), _MAX_REL, np.int32)
    for w in range(_NUM_TILES):
        for li in range(2):
            i = 2 * w + li
            for c in range(1, 64):
                j = c - 1
                idx_r[w, li * 64 + c] = i // 8 - j // 8 + _MAX_REL
                idx_f[w, li * 64 + c] = i % 8 - j % 8 + _MAX_REL
    return idx_r, idx_f


_IDXR_HOST, _IDXF_HOST = _host_indices()


def _tanh2(g):
    g = jnp.clip(g, -20.0, 20.0)
    e = jnp.exp(g * -2.0)
    return (2.0 - 2.0 * e) / (1.0 + e)


def _body(table_hbm, idxr_hbm, idxf_hbm, out_hbm, table_v, idxr_v, idxf_v, buf):
    wid = lax.axis_index("s") * 2 + lax.axis_index("c")
    pltpu.sync_copy(table_hbm, table_v)
    pltpu.sync_copy(idxr_hbm.at[wid], idxr_v)
    pltpu.sync_copy(idxf_hbm.at[wid], idxf_v)

    lane = lax.iota(jnp.int32, 16)
    zeros = jnp.zeros((16,), jnp.float32)

    for k in range(8):
        li = k // 4
        c0 = (k % 4) * 16
        rv = idxr_v[pl.ds(k * 16, 16)]
        fv = idxf_v[pl.ds(k * 16, 16)]
        for h in range(_NUM_HEADS):
            g = plsc.load_gather(
                table_v, [rv, fv, jnp.full((16,), h, jnp.int32)])
            buf[h, 1 + li, pl.ds(c0, 16)] = _tanh2(g)

    for li in range(2):
        plsc.store_scatter(
            buf, [lane, jnp.full((16,), 1 + li, jnp.int32),
                  jnp.zeros((16,), jnp.int32)], zeros)
        i = 2 * wid + li
        row = table_v[i // 8, lax.rem(i, 8), :]
        plsc.store_scatter(
            buf, [lane, jnp.full((16,), 1 + li, jnp.int32),
                  jnp.full((16,), 64, jnp.int32)], _tanh2(row))

    @pl.when(wid == 0)
    def _():
        for h in range(_NUM_HEADS):
            for c0 in range(0, 64, 16):
                buf[h, 0, pl.ds(c0, 16)] = zeros
        plsc.store_scatter(
            buf, [lane, jnp.zeros((16,), jnp.int32),
                  jnp.full((16,), 64, jnp.int32)], zeros)
        pltpu.sync_copy(buf, out_hbm.at[0, :, pl.ds(0, 3), :])

    @pl.when(wid != 0)
    def _():
        pltpu.sync_copy(buf.at[:, 1:, :],
                        out_hbm.at[0, :, pl.ds(2 * wid + 1, 2), :])


@jax.jit
def _run(table):
    mesh = plsc.VectorSubcoreMesh(core_axis_name="c", subcore_axis_name="s")
    return pl.kernel(
        _body,
        out_type=jax.ShapeDtypeStruct((1, _NUM_HEADS, 65, 65), jnp.float32),
        mesh=mesh,
        compiler_params=pltpu.CompilerParams(use_tc_tiling_on_sc=False,
                                             needs_layout_passes=False,
                                             skip_device_barrier=True),
        scratch_types=[
            pltpu.VMEM((_NUM_BUCKETS, _NUM_BUCKETS, _NUM_HEADS), jnp.float32),
            pltpu.VMEM((_IDX_LEN,), jnp.int32),
            pltpu.VMEM((_IDX_LEN,), jnp.int32),
            pltpu.VMEM((_NUM_HEADS, 3, 65), jnp.float32),
        ],
    )(table, jnp.asarray(_IDXR_HOST), jnp.asarray(_IDXF_HOST))


def kernel(relative_bias_table, seq_len):
    del seq_len
    return _run(relative_bias_table)


# speedup vs baseline: 1.0576x; 1.0576x over previous
import jax
import jax.numpy as jnp
import numpy as np
from jax import lax
from jax.experimental import pallas as pl
from jax.experimental.pallas import tpu as pltpu
from jax.experimental.pallas import tpu_sc as plsc

_MAX_REL = 7
_NUM_BUCKETS = 2 * _MAX_REL + 1
_NUM_HEADS = 16
_NUM_TILES = 16
_ROWS_PER_TILE = 4
_IDX_LEN = _ROWS_PER_TILE * 64


def _host_indices():
    idx_r = np.full((_NUM_TILES, _IDX_LEN), _MAX_REL, np.int32)
    idx_f = np.full((_NUM_TILES, _IDX_LEN), _MAX_REL, np.int32)
    for w in range(_NUM_TILES):
        for li in range(_ROWS_PER_TILE):
            i = _ROWS_PER_TILE * w + li
            for c in range(1, 64):
                j = c - 1
                idx_r[w, li * 64 + c] = i // 8 - j // 8 + _MAX_REL
                idx_f[w, li * 64 + c] = i % 8 - j % 8 + _MAX_REL
    return idx_r, idx_f


_IDXR_HOST, _IDXF_HOST = _host_indices()


def _tanh2(g):
    g = jnp.clip(g, -20.0, 20.0)
    e = jnp.exp(g * -2.0)
    return (2.0 - 2.0 * e) / (1.0 + e)


def _body(table_hbm, idxr_hbm, idxf_hbm, out_hbm, table_v, idxr_v, idxf_v, buf):
    wid = lax.axis_index("s")
    pltpu.sync_copy(table_hbm, table_v)
    pltpu.sync_copy(idxr_hbm.at[wid], idxr_v)
    pltpu.sync_copy(idxf_hbm.at[wid], idxf_v)

    lane = lax.iota(jnp.int32, 16)
    zeros = jnp.zeros((16,), jnp.float32)

    def chunk(k, carry):
        li = 1 + k // 4
        c0 = pl.multiple_of((k % 4) * 16, 16)
        rv = idxr_v[pl.ds(k * 16, 16)]
        fv = idxf_v[pl.ds(k * 16, 16)]

        def head(h, c):
            g = plsc.load_gather(
                table_v, [rv, fv, jnp.full((16,), h, jnp.int32)])
            buf[h, li, pl.ds(c0, 16)] = _tanh2(g)
            return c

        return lax.fori_loop(0, _NUM_HEADS, head, carry)

    lax.fori_loop(0, 16, chunk, 0)

    def edges(li, carry):
        plsc.store_scatter(
            buf, [lane, jnp.full((16,), 1 + li, jnp.int32),
                  jnp.zeros((16,), jnp.int32)], zeros)
        i = _ROWS_PER_TILE * wid + li
        row = table_v[i // 8, lax.rem(i, 8), :]
        plsc.store_scatter(
            buf, [lane, jnp.full((16,), 1 + li, jnp.int32),
                  jnp.full((16,), 64, jnp.int32)], _tanh2(row))
        return carry

    lax.fori_loop(0, _ROWS_PER_TILE, edges, 0)

    @pl.when(wid == 0)
    def _():
        def zrow(h, carry):
            for c0 in range(0, 64, 16):
                buf[h, 0, pl.ds(c0, 16)] = zeros
            return carry

        lax.fori_loop(0, _NUM_HEADS, zrow, 0)
        plsc.store_scatter(
            buf, [lane, jnp.zeros((16,), jnp.int32),
                  jnp.full((16,), 64, jnp.int32)], zeros)
        pltpu.sync_copy(buf, out_hbm.at[0, :, pl.ds(0, 5), :])

    @pl.when(wid != 0)
    def _():
        pltpu.sync_copy(
            buf.at[:, 1:, :],
            out_hbm.at[0, :, pl.ds(_ROWS_PER_TILE * wid + 1, _ROWS_PER_TILE),
                       :])


@jax.jit
def _run(table):
    mesh = plsc.VectorSubcoreMesh(core_axis_name="c", subcore_axis_name="s",
                                  num_cores=1)
    return pl.kernel(
        _body,
        out_type=jax.ShapeDtypeStruct((1, _NUM_HEADS, 65, 65), jnp.float32),
        mesh=mesh,
        compiler_params=pltpu.CompilerParams(use_tc_tiling_on_sc=False,
                                             needs_layout_passes=False,
                                             skip_device_barrier=True),
        scratch_types=[
            pltpu.VMEM((_NUM_BUCKETS, _NUM_BUCKETS, _NUM_HEADS), jnp.float32),
            pltpu.VMEM((_IDX_LEN,), jnp.int32),
            pltpu.VMEM((_IDX_LEN,), jnp.int32),
            pltpu.VMEM((_NUM_HEADS, 1 + _ROWS_PER_TILE, 65), jnp.float32),
        ],
    )(table, jnp.asarray(_IDXR_HOST), jnp.asarray(_IDXF_HOST))


def kernel(relative_bias_table, seq_len):
    del seq_len
    return _run(relative_bias_table)


# speedup vs baseline: 1.1061x; 1.0459x over previous
import jax
import jax.numpy as jnp
from jax import lax
from jax.experimental import pallas as pl
from jax.experimental.layout import Format, Layout
from jax.experimental.pallas import tpu as pltpu
from jax.experimental.pallas import tpu_sc as plsc

_MAX_REL = 7
_NUM_BUCKETS = 2 * _MAX_REL + 1
_NUM_HEADS = 16
_ROWS_PER_TILE = 4


def _tanh2(g):
    g = jnp.clip(g, -20.0, 20.0)
    e = jnp.exp(g * -2.0)
    return (2.0 - 2.0 * e) / (1.0 + e)


def _body(table_hbm, out_hbm, table_v, buf, sem):
    wid = lax.axis_index("s")
    copy = pltpu.async_copy(table_hbm, table_v, sem)

    lane = lax.iota(jnp.int32, 16)
    zeros = jnp.zeros((16,), jnp.float32)

    @pl.when(wid == 0)
    def _():
        def zrow(h, carry):
            for c0 in range(0, 64, 16):
                buf[h, 0, pl.ds(c0, 16)] = zeros
            return carry

        lax.fori_loop(0, _NUM_HEADS, zrow, 0)
        plsc.store_scatter(
            buf, [lane, jnp.zeros((16,), jnp.int32),
                  jnp.full((16,), 64, jnp.int32)], zeros)

    copy.wait()

    def chunk(k, carry):
        li = 1 + k // 4
        i = _ROWS_PER_TILE * wid + k // 4
        c0 = (k % 4) * 16
        c = c0 + lane
        j = jnp.maximum(c - 1, 0)
        rv = jnp.full((16,), i // 8 + _MAX_REL, jnp.int32) - (
            lax.shift_right_logical(j, 3))
        fv = jnp.full((16,), lax.rem(i, 8) + _MAX_REL, jnp.int32) - (
            jnp.bitwise_and(j, 7))
        for h in range(_NUM_HEADS):
            g = plsc.load_gather(
                table_v, [rv, fv, jnp.full((16,), h, jnp.int32)])
            buf[h, li, pl.ds(c0, 16)] = _tanh2(g)
        return carry

    lax.fori_loop(0, 4 * _ROWS_PER_TILE, chunk, 0)

    def edges(li, carry):
        plsc.store_scatter(
            buf, [lane, jnp.full((16,), 1 + li, jnp.int32),
                  jnp.zeros((16,), jnp.int32)], zeros)
        i = _ROWS_PER_TILE * wid + li
        row = table_v[i // 8, lax.rem(i, 8), :]
        plsc.store_scatter(
            buf, [lane, jnp.full((16,), 1 + li, jnp.int32),
                  jnp.full((16,), 64, jnp.int32)], _tanh2(row))
        return carry

    lax.fori_loop(0, _ROWS_PER_TILE, edges, 0)

    @pl.when(wid == 0)
    def _():
        pltpu.sync_copy(buf, out_hbm.at[0, :, pl.ds(0, 1 + _ROWS_PER_TILE), :])

    @pl.when(wid != 0)
    def _():
        pltpu.sync_copy(
            buf.at[:, 1:, :],
            out_hbm.at[0, :, pl.ds(_ROWS_PER_TILE * wid + 1, _ROWS_PER_TILE),
                       :])


@jax.jit
def _run(table):
    mesh = plsc.VectorSubcoreMesh(core_axis_name="c", subcore_axis_name="s",
                                  num_cores=1)
    return pl.kernel(
        _body,
        out_type=jax.ShapeDtypeStruct((1, _NUM_HEADS, 65, 65), jnp.float32),
        mesh=mesh,
        compiler_params=pltpu.CompilerParams(use_tc_tiling_on_sc=False,
                                             needs_layout_passes=False,
                                             skip_device_barrier=True),
        scratch_types=[
            pltpu.VMEM((_NUM_BUCKETS, _NUM_BUCKETS, _NUM_HEADS), jnp.float32),
            pltpu.VMEM((_NUM_HEADS, 1 + _ROWS_PER_TILE, 65), jnp.float32),
            pltpu.SemaphoreType.DMA,
        ],
    )(table)


_RUN_LINEAR = None


def _run_linear_out():
    global _RUN_LINEAR
    if _RUN_LINEAR is None:
        fmt = Format(
            Layout(major_to_minor=(0, 1, 2, 3), tiling=()),
            jax.sharding.SingleDeviceSharding(jax.devices()[0]))
        _RUN_LINEAR = jax.jit(_run.__wrapped__, out_shardings=fmt)
    return _RUN_LINEAR


def kernel(relative_bias_table, seq_len):
    del seq_len
    return _run_linear_out()(relative_bias_table)


# speedup vs baseline: 1.1838x; 1.0702x over previous
import jax
import jax.numpy as jnp
from jax import lax
from jax.experimental import pallas as pl
from jax.experimental.layout import Format, Layout
from jax.experimental.pallas import tpu as pltpu
from jax.experimental.pallas import tpu_sc as plsc

_MAX_REL = 7
_NUM_BUCKETS = 2 * _MAX_REL + 1
_NUM_HEADS = 16
_ROWS_PER_TILE = 4


def _tanh2(g):
    g = jnp.clip(g, -20.0, 20.0)
    e = jnp.exp(g * -2.0)
    return (2.0 - 2.0 * e) / (1.0 + e)


def _body(table_hbm, out_hbm, table_v, buf, sem):
    wid = lax.axis_index("s")
    copy = pltpu.async_copy(table_hbm, table_v, sem)

    lane = lax.iota(jnp.int32, 16)
    zeros = jnp.zeros((16,), jnp.float32)

    @pl.when(wid == 0)
    def _():
        def zrow(h, carry):
            for c0 in range(0, 64, 16):
                buf[h, 0, pl.ds(c0, 16)] = zeros
            return carry

        lax.fori_loop(0, _NUM_HEADS, zrow, 0)
        plsc.store_scatter(
            buf, [lane, jnp.zeros((16,), jnp.int32),
                  jnp.full((16,), 64, jnp.int32)], zeros)

    copy.wait()

    @plsc.parallel_loop(0, 4 * _ROWS_PER_TILE, step=1, unroll=4)
    def chunk(k):
        li = 1 + k // 4
        i = _ROWS_PER_TILE * wid + k // 4
        c0 = (k % 4) * 16
        c = c0 + lane
        j = jnp.maximum(c - 1, 0)
        rv = jnp.full((16,), i // 8 + _MAX_REL, jnp.int32) - (
            lax.shift_right_logical(j, 3))
        fv = jnp.full((16,), lax.rem(i, 8) + _MAX_REL, jnp.int32) - (
            jnp.bitwise_and(j, 7))
        for h in range(_NUM_HEADS):
            g = plsc.load_gather(
                table_v, [rv, fv, jnp.full((16,), h, jnp.int32)])
            buf[h, li, pl.ds(c0, 16)] = _tanh2(g)

    def edges(li, carry):
        plsc.store_scatter(
            buf, [lane, jnp.full((16,), 1 + li, jnp.int32),
                  jnp.zeros((16,), jnp.int32)], zeros)
        i = _ROWS_PER_TILE * wid + li
        row = table_v[i // 8, lax.rem(i, 8), :]
        plsc.store_scatter(
            buf, [lane, jnp.full((16,), 1 + li, jnp.int32),
                  jnp.full((16,), 64, jnp.int32)], _tanh2(row))
        return carry

    lax.fori_loop(0, _ROWS_PER_TILE, edges, 0)

    @pl.when(wid == 0)
    def _():
        pltpu.sync_copy(buf, out_hbm.at[0, :, pl.ds(0, 1 + _ROWS_PER_TILE), :])

    @pl.when(wid != 0)
    def _():
        pltpu.sync_copy(
            buf.at[:, 1:, :],
            out_hbm.at[0, :, pl.ds(_ROWS_PER_TILE * wid + 1, _ROWS_PER_TILE),
                       :])


@jax.jit
def _run(table):
    mesh = plsc.VectorSubcoreMesh(core_axis_name="c", subcore_axis_name="s",
                                  num_cores=1)
    return pl.kernel(
        _body,
        out_type=jax.ShapeDtypeStruct((1, _NUM_HEADS, 65, 65), jnp.float32),
        mesh=mesh,
        compiler_params=pltpu.CompilerParams(use_tc_tiling_on_sc=False,
                                             needs_layout_passes=False,
                                             skip_device_barrier=True),
        scratch_types=[
            pltpu.VMEM((_NUM_BUCKETS, _NUM_BUCKETS, _NUM_HEADS), jnp.float32),
            pltpu.VMEM((_NUM_HEADS, 1 + _ROWS_PER_TILE, 65), jnp.float32),
            pltpu.SemaphoreType.DMA,
        ],
    )(table)


_RUN_LINEAR = None


def _run_linear_out():
    global _RUN_LINEAR
    if _RUN_LINEAR is None:
        fmt = Format(
            Layout(major_to_minor=(0, 1, 2, 3), tiling=()),
            jax.sharding.SingleDeviceSharding(jax.devices()[0]))
        _RUN_LINEAR = jax.jit(_run.__wrapped__, out_shardings=fmt)
    return _RUN_LINEAR


def kernel(relative_bias_table, seq_len):
    del seq_len
    return _run_linear_out()(relative_bias_table)
